# baseline (speedup 1.0000x reference)
import functools

import jax
import jax.numpy as jnp
from jax.experimental import pallas as pl

_ROWS = 2048


def _embed_block(feat_ref, mask_ref, gidx_ref, wt_ref, bias_ref, cls_ref, out_ref):
    i = pl.program_id(0)
    lin = jnp.dot(feat_ref[...], wt_ref[...], preferred_element_type=jnp.float32)
    lin = (lin + bias_ref[...]) * mask_ref[...]
    rid = i * _ROWS + jax.lax.broadcasted_iota(jnp.int32, (_ROWS, 1), 0)
    is_cls = (rid == gidx_ref[...]).any(axis=1, keepdims=True)
    out_ref[...] = jnp.where(is_cls, cls_ref[...], lin)


def kernel(feat, amask, g_idx, b_idx, W, bias, cls_token):
    n, token_dim = feat.shape
    emb_dim = W.shape[0]
    nb = g_idx.shape[0]
    maskf = amask.reshape(n, 1).astype(jnp.float32)
    out = pl.pallas_call(
        _embed_block,
        grid=(n // _ROWS,),
        in_specs=[
            pl.BlockSpec((_ROWS, token_dim), lambda i: (i, 0)),
            pl.BlockSpec((_ROWS, 1), lambda i: (i, 0)),
            pl.BlockSpec((1, nb), lambda i: (0, 0)),
            pl.BlockSpec((token_dim, emb_dim), lambda i: (0, 0)),
            pl.BlockSpec((1, emb_dim), lambda i: (0, 0)),
            pl.BlockSpec((1, emb_dim), lambda i: (0, 0)),
        ],
        out_specs=pl.BlockSpec((_ROWS, emb_dim), lambda i: (i, 0)),
        out_shape=jax.ShapeDtypeStruct((n, emb_dim), jnp.float32),
    )(
        feat,
        maskf,
        g_idx.astype(jnp.int32).reshape(1, nb),
        W.T,
        bias.reshape(1, emb_dim),
        cls_token.reshape(1, emb_dim),
    )
    return (out, amask, g_idx, b_idx)
